# baseline (speedup 1.0000x reference)
import functools

import jax
import jax.numpy as jnp
from jax import lax
from jax.experimental import pallas as pl
from jax.experimental.pallas import tpu as pltpu
from jax.experimental.pallas import tpu_sc as plsc

LQP = 16
K = 20


def _gather_rows_sc(table, idx_q, idx_d, n_split, offs):
    D = table.shape[1]
    q_off, d_off = offs
    info = plsc.get_sparse_core_info()
    NC, NS = info.num_cores, info.num_subcores
    NW = NC * NS
    C = 64
    NBUF = max(nb for nb in (8, 4, 2, 1)
               if (n_split[0] // NW // C) % nb == 0
               and (n_split[1] // NW // C) % nb == 0)

    mesh = plsc.VectorSubcoreMesh(core_axis_name="c", subcore_axis_name="s")

    NQ, ND = n_split
    per_w_q = NQ // NW
    per_w_d = ND // NW
    per_w = per_w_q + per_w_d

    @functools.partial(
        pl.kernel,
        mesh=mesh,
        out_type=(jax.ShapeDtypeStruct((NQ, D), jnp.float32),
                  jax.ShapeDtypeStruct((ND, D), jnp.float32)),
        scratch_types=[pltpu.VMEM((per_w,), jnp.int32)]
        + [pltpu.VMEM((C, D), jnp.float32) for _ in range(NBUF)]
        + [pltpu.SemaphoreType.DMA for _ in range(NBUF)],
    )
    def k(table_hbm, idxq_hbm, idxd_hbm, outq_hbm, outd_hbm, idx_v,
          *bufs_sems):
        bufs = bufs_sems[:NBUF]
        sems = bufs_sems[NBUF:]
        wid = lax.axis_index("s") * NC + lax.axis_index("c")
        pltpu.sync_copy(idxq_hbm.at[pl.ds(q_off + wid * per_w_q, per_w_q)],
                        idx_v.at[pl.ds(0, per_w_q)])
        pltpu.sync_copy(idxd_hbm.at[pl.ds(d_off + wid * per_w_d, per_w_d)],
                        idx_v.at[pl.ds(per_w_q, per_w_d)])

        def run(v_off, out_hbm, out_base, nch):
            def fire(i, p):
                pltpu.async_copy(
                    table_hbm.at[idx_v.at[pl.ds(v_off + i * C, C)]],
                    bufs[p], sems[p])

            for p in range(NBUF):
                fire(p, p)

            def body(t, carry):
                for p in range(NBUF):
                    i = t * NBUF + p
                    pltpu.make_async_copy(
                        table_hbm.at[idx_v.at[pl.ds(v_off + i * C, C)]],
                        bufs[p], sems[p]).wait()
                    pltpu.sync_copy(bufs[p],
                                    out_hbm.at[pl.ds(out_base + i * C, C)])

                    @pl.when(i + NBUF < nch)
                    def _():
                        fire(i + NBUF, p)
                return carry

            lax.fori_loop(0, nch // NBUF, body, 0)

        run(0, outq_hbm, wid * per_w_q, per_w_q // C)
        run(per_w_q, outd_hbm, wid * per_w_d, per_w_d // C)

    return k(table, idx_q, idx_d)


def _tc_body(q_ref, d_ref, tl_ref, loc_ref, dist_ref,
             attn_ref, w1_ref, b1_ref, w2_ref, b2_ref, w3_ref, b3_ref,
             outw_ref, outb_ref, m2_ref, lat2_ref, lon2_ref, ball_ref,
             pred_ref, mm_s, tk_s, *, TB, LR, NLOC):
    f32 = jnp.float32

    eps = lax.broadcasted_iota(jnp.int32, (LQP, LR), 1).astype(f32) * 1e-6

    for b in range(TB):
        qb = q_ref[b * LQP:(b + 1) * LQP, :]
        nq = jnp.sqrt(jnp.sum(qb * qb, axis=1, keepdims=True))
        qn = qb / jnp.maximum(nq, 1e-12)
        db = d_ref[b * LR:(b + 1) * LR, :]
        nd = jnp.sqrt(jnp.sum(db * db, axis=1, keepdims=True))
        dn = db / jnp.maximum(nd, 1e-12)
        mm = lax.dot_general(qn, dn, (((1,), (1,)), ((), ())),
                             preferred_element_type=f32)
        mm_s[b * LQP:(b + 1) * LQP, :] = mm + eps

    R = TB * LQP

    for k in range(K):
        xv = mm_s[:, :]
        m = jnp.max(xv, axis=1, keepdims=True)
        mm_s[:, :] = jnp.where(xv == m, -jnp.inf, xv)
        tk_s[:, k:k + 1] = m

    h = jnp.tanh(lax.dot_general(tk_s[:, :], w1_ref[:, :],
                                 (((1,), (0,)), ((), ())),
                                 preferred_element_type=f32) + b1_ref[:, :])
    h = jnp.tanh(lax.dot_general(h, w2_ref[:, :], (((1,), (0,)), ((), ())),
                                 preferred_element_type=f32) + b2_ref[:, :])
    h3 = jnp.tanh(lax.dot_general(h, w3_ref[:, :], (((1,), (0,)), ((), ())),
                                  preferred_element_type=f32) + b3_ref[0, 0])

    logits = lax.dot_general(q_ref[:, :], attn_ref[:, :],
                             (((1,), (0,)), ((), ())),
                             preferred_element_type=f32)
    mask = tl_ref[:, :] == 0.0
    e = jnp.where(mask, 0.0, jnp.exp(logits))
    rowi = lax.broadcasted_iota(jnp.int32, (TB, R), 0)
    colj = lax.broadcasted_iota(jnp.int32, (TB, R), 1)
    S = jnp.where(colj // LQP == rowi, 1.0, 0.0)
    ssum = lax.dot_general(S, e, (((1,), (0,)), ((), ())),
                           preferred_element_type=f32)
    denom = lax.dot_general(S, ssum, (((0,), (0,)), ((), ())),
                            preferred_element_type=f32)
    probs = e / denom
    xw = lax.dot_general(S, probs * h3, (((1,), (0,)), ((), ())),
                         preferred_element_type=f32)
    xx = jnp.tanh(xw * outw_ref[0, 0] + outb_ref[0, 0])

    P = lax.dot_general(q_ref[:, :], m2_ref[:, :], (((1,), (0,)), ((), ())),
                        preferred_element_type=f32)
    rmod = lax.broadcasted_iota(jnp.int32, (R, 2 * LQP), 0) % LQP
    jmod = lax.broadcasted_iota(jnp.int32, (R, 2 * LQP), 1) % LQP
    PE = jnp.where(jmod == rmod, P, 0.0)
    hsel = jnp.where(
        lax.broadcasted_iota(jnp.int32, (2 * LQP, 2), 0) // LQP
        == lax.broadcasted_iota(jnp.int32, (2 * LQP, 2), 1), 1.0, 0.0)
    PE2 = lax.dot_general(PE, hsel, (((1,), (0,)), ((), ())),
                          preferred_element_type=f32)
    wgeo = lax.dot_general(S, PE2, (((1,), (0,)), ((), ())),
                           preferred_element_type=f32)
    gi = lax.broadcasted_iota(jnp.int32, (TB, NLOC), 1)
    oh_lat = jnp.where(gi == loc_ref[:, 0:1], 1.0, 0.0)
    oh_lon = jnp.where(gi == loc_ref[:, 1:2], 1.0, 0.0)
    wgeo = wgeo + lax.dot_general(oh_lat, lat2_ref[:, :], (((1,), (0,)), ((), ())),
                                  preferred_element_type=f32)
    wgeo = wgeo + lax.dot_general(oh_lon, lon2_ref[:, :], (((1,), (0,)), ((), ())),
                                  preferred_element_type=f32)
    wgeo = wgeo + ball_ref[:, :]

    pred_ref[:, :] = wgeo[:, 0:1] * xx + wgeo[:, 1:2] * dist_ref[:, :]


def kernel(text_left, text_right, location_left, distance, emb_table, attn_w,
           mlp_w1, mlp_b1, mlp_w2, mlp_b2, mlp_w3, mlp_b3, out_w, out_b,
           lat_table, lon_table, lin5_w, lin5_b, lin3_w, lin3_b, lin4_w, lin4_b):
    B, LQ = text_left.shape
    LR = text_right.shape[1]
    D = emb_table.shape[1]
    NLOC = lat_table.shape[0]
    H3 = lin3_w.shape[0]
    GEOW = lin5_w.shape[0]
    f32 = jnp.float32

    tl_pad = jnp.pad(text_left, ((0, 0), (0, LQP - LQ)))

    NS = 4
    Bs = B // NS
    idx_q = tl_pad.reshape(-1).astype(jnp.int32)
    idx_d = text_right.reshape(-1).astype(jnp.int32)
    gathered = []
    for si in range(NS):
        gathered.append(
            _gather_rows_sc(emb_table, idx_q, idx_d, (Bs * LQP, Bs * LR),
                            (si * Bs * LQP, si * Bs * LR)))

    attn_wT = attn_w.T.astype(f32)
    w1T = mlp_w1.T.astype(f32)
    b1 = mlp_b1.reshape(1, -1).astype(f32)
    w2T = mlp_w2.T.astype(f32)
    b2 = mlp_b2.reshape(1, -1).astype(f32)
    w3T = mlp_w3.T.astype(f32)
    b3 = mlp_b3.reshape(1, 1).astype(f32)
    outw = out_w.reshape(1, 1).astype(f32)
    outb = out_b.reshape(1, 1).astype(f32)

    A1 = lin4_w[:, :LQ * H3].reshape(2, LQ, H3)
    M3 = jnp.einsum('jd,clj->cld', lin3_w, A1)
    M3 = jnp.pad(M3, ((0, 0), (0, LQP - LQ), (0, 0)))
    M2 = M3.transpose(2, 0, 1).reshape(D, 2 * LQP).astype(f32)
    A2 = lin4_w[:, LQ * H3:LQ * H3 + GEOW]
    A3 = lin4_w[:, LQ * H3 + GEOW:LQ * H3 + 2 * GEOW]
    lat2 = (lat_table @ (lin5_w.T @ A2.T)).astype(f32)
    lon2 = (lon_table @ (lin5_w.T @ A3.T)).astype(f32)
    b_all = (lin4_b
             + jnp.einsum('j,clj->c', lin3_b, A1)
             + lin5_b @ A2.T + lin5_b @ A3.T).reshape(1, 2).astype(f32)

    tlf = tl_pad.astype(f32).reshape(B * LQP, 1)
    locf = location_left.astype(jnp.int32)
    dist2 = distance.astype(f32).reshape(B, 1)

    TB = min(64, Bs)
    G = Bs // TB

    body = functools.partial(_tc_body, TB=TB, LR=LR, NLOC=NLOC)

    in_specs = [
            pl.BlockSpec((TB * LQP, D), lambda g: (g, 0)),
            pl.BlockSpec((TB * LR, D), lambda g: (g, 0)),
            pl.BlockSpec((TB * LQP, 1), lambda g: (g, 0)),
            pl.BlockSpec((TB, 2), lambda g: (g, 0)),
            pl.BlockSpec((TB, 1), lambda g: (g, 0)),
            pl.BlockSpec((D, 1), lambda g: (0, 0)),
            pl.BlockSpec((K, 128), lambda g: (0, 0)),
            pl.BlockSpec((1, 128), lambda g: (0, 0)),
            pl.BlockSpec((128, 128), lambda g: (0, 0)),
            pl.BlockSpec((1, 128), lambda g: (0, 0)),
            pl.BlockSpec((128, 1), lambda g: (0, 0)),
            pl.BlockSpec((1, 1), lambda g: (0, 0)),
            pl.BlockSpec((1, 1), lambda g: (0, 0)),
            pl.BlockSpec((1, 1), lambda g: (0, 0)),
            pl.BlockSpec((D, 2 * LQP), lambda g: (0, 0)),
            pl.BlockSpec((NLOC, 2), lambda g: (0, 0)),
            pl.BlockSpec((NLOC, 2), lambda g: (0, 0)),
            pl.BlockSpec((1, 2), lambda g: (0, 0)),
    ]

    preds = []
    for si in range(NS):
        q_pad, d_rows = gathered[si]
        preds.append(pl.pallas_call(
            body,
            grid=(G,),
            in_specs=in_specs,
            out_specs=pl.BlockSpec((TB, 1), lambda g: (g, 0)),
            out_shape=jax.ShapeDtypeStruct((Bs, 1), f32),
            scratch_shapes=[
                pltpu.VMEM((TB * LQP, LR), f32),
                pltpu.VMEM((TB * LQP, K), f32),
            ],
        )(q_pad, d_rows,
          tlf[si * Bs * LQP:(si + 1) * Bs * LQP],
          locf[si * Bs:(si + 1) * Bs],
          dist2[si * Bs:(si + 1) * Bs],
          attn_wT, w1T, b1, w2T, b2,
          w3T, b3, outw, outb, M2, lat2, lon2, b_all))

    return jnp.concatenate(preds, axis=0)


# speedup vs baseline: 1.0033x; 1.0033x over previous
import functools

import jax
import jax.numpy as jnp
from jax import lax
from jax.experimental import pallas as pl
from jax.experimental.pallas import tpu as pltpu
from jax.experimental.pallas import tpu_sc as plsc

LQP = 16
K = 20


def _gather_rows_sc(table, idx_all, n_split):
    D = table.shape[1]
    info = plsc.get_sparse_core_info()
    NC, NS = info.num_cores, info.num_subcores
    NW = NC * NS
    C = 64
    NBUF = max(nb for nb in (8, 4, 2, 1)
               if (n_split[0] // NW // C) % nb == 0
               and (n_split[1] // NW // C) % nb == 0)

    mesh = plsc.VectorSubcoreMesh(core_axis_name="c", subcore_axis_name="s")

    NQ, ND = n_split
    per_w_q = NQ // NW
    per_w_d = ND // NW
    per_w = per_w_q + per_w_d

    @functools.partial(
        pl.kernel,
        mesh=mesh,
        out_type=(jax.ShapeDtypeStruct((NQ, D), jnp.float32),
                  jax.ShapeDtypeStruct((ND, D), jnp.float32)),
        scratch_types=[pltpu.VMEM((per_w,), jnp.int32)]
        + [pltpu.VMEM((C, D), jnp.float32) for _ in range(NBUF)]
        + [pltpu.SemaphoreType.DMA for _ in range(NBUF)],
    )
    def k(table_hbm, idx_hbm, outq_hbm, outd_hbm, idx_v, *bufs_sems):
        bufs = bufs_sems[:NBUF]
        sems = bufs_sems[NBUF:]
        wid = lax.axis_index("s") * NC + lax.axis_index("c")
        pltpu.sync_copy(idx_hbm.at[pl.ds(wid * per_w_q, per_w_q)],
                        idx_v.at[pl.ds(0, per_w_q)])
        pltpu.sync_copy(idx_hbm.at[pl.ds(NQ + wid * per_w_d, per_w_d)],
                        idx_v.at[pl.ds(per_w_q, per_w_d)])

        def run(v_off, out_hbm, out_base, nch):
            def fire(i, p):
                pltpu.async_copy(
                    table_hbm.at[idx_v.at[pl.ds(v_off + i * C, C)]],
                    bufs[p], sems[p])

            for p in range(NBUF):
                fire(p, p)

            def body(t, carry):
                for p in range(NBUF):
                    i = t * NBUF + p
                    pltpu.make_async_copy(
                        table_hbm.at[idx_v.at[pl.ds(v_off + i * C, C)]],
                        bufs[p], sems[p]).wait()
                    pltpu.sync_copy(bufs[p],
                                    out_hbm.at[pl.ds(out_base + i * C, C)])

                    @pl.when(i + NBUF < nch)
                    def _():
                        fire(i + NBUF, p)
                return carry

            lax.fori_loop(0, nch // NBUF, body, 0)

        run(0, outq_hbm, wid * per_w_q, per_w_q // C)
        run(per_w_q, outd_hbm, wid * per_w_d, per_w_d // C)

    return k(table, idx_all)


def _tc_body(q_ref, d_ref, tl_ref, loc_ref, dist_ref,
             attn_ref, w1_ref, b1_ref, w2_ref, b2_ref, w3_ref, b3_ref,
             outw_ref, outb_ref, m2_ref, lat2_ref, lon2_ref, ball_ref,
             pred_ref, mm_s, tk_s, *, TB, LR, NLOC):
    f32 = jnp.float32

    eps = lax.broadcasted_iota(jnp.int32, (LQP, LR), 1).astype(f32) * 1e-6

    for b in range(TB):
        qb = q_ref[b * LQP:(b + 1) * LQP, :]
        nq = jnp.sqrt(jnp.sum(qb * qb, axis=1, keepdims=True))
        qn = qb / jnp.maximum(nq, 1e-12)
        db = d_ref[b * LR:(b + 1) * LR, :]
        nd = jnp.sqrt(jnp.sum(db * db, axis=1, keepdims=True))
        dn = db / jnp.maximum(nd, 1e-12)
        mm = lax.dot_general(qn, dn, (((1,), (1,)), ((), ())),
                             preferred_element_type=f32)
        mm_s[b * LQP:(b + 1) * LQP, :] = mm + eps

    R = TB * LQP

    for k in range(K):
        xv = mm_s[:, :]
        m = jnp.max(xv, axis=1, keepdims=True)
        mm_s[:, :] = jnp.where(xv == m, -jnp.inf, xv)
        tk_s[:, k:k + 1] = m

    h = jnp.tanh(lax.dot_general(tk_s[:, :], w1_ref[:, :],
                                 (((1,), (0,)), ((), ())),
                                 preferred_element_type=f32) + b1_ref[:, :])
    h = jnp.tanh(lax.dot_general(h, w2_ref[:, :], (((1,), (0,)), ((), ())),
                                 preferred_element_type=f32) + b2_ref[:, :])
    h3 = jnp.tanh(lax.dot_general(h, w3_ref[:, :], (((1,), (0,)), ((), ())),
                                  preferred_element_type=f32) + b3_ref[0, 0])

    logits = lax.dot_general(q_ref[:, :], attn_ref[:, :],
                             (((1,), (0,)), ((), ())),
                             preferred_element_type=f32)
    mask = tl_ref[:, :] == 0.0
    e = jnp.where(mask, 0.0, jnp.exp(logits))
    rowi = lax.broadcasted_iota(jnp.int32, (TB, R), 0)
    colj = lax.broadcasted_iota(jnp.int32, (TB, R), 1)
    S = jnp.where(colj // LQP == rowi, 1.0, 0.0)
    ssum = lax.dot_general(S, e, (((1,), (0,)), ((), ())),
                           preferred_element_type=f32)
    denom = lax.dot_general(S, ssum, (((0,), (0,)), ((), ())),
                            preferred_element_type=f32)
    probs = e / denom
    xw = lax.dot_general(S, probs * h3, (((1,), (0,)), ((), ())),
                         preferred_element_type=f32)
    xx = jnp.tanh(xw * outw_ref[0, 0] + outb_ref[0, 0])

    P = lax.dot_general(q_ref[:, :], m2_ref[:, :], (((1,), (0,)), ((), ())),
                        preferred_element_type=f32)
    rmod = lax.broadcasted_iota(jnp.int32, (R, 2 * LQP), 0) % LQP
    jmod = lax.broadcasted_iota(jnp.int32, (R, 2 * LQP), 1) % LQP
    PE = jnp.where(jmod == rmod, P, 0.0)
    hsel = jnp.where(
        lax.broadcasted_iota(jnp.int32, (2 * LQP, 2), 0) // LQP
        == lax.broadcasted_iota(jnp.int32, (2 * LQP, 2), 1), 1.0, 0.0)
    PE2 = lax.dot_general(PE, hsel, (((1,), (0,)), ((), ())),
                          preferred_element_type=f32)
    wgeo = lax.dot_general(S, PE2, (((1,), (0,)), ((), ())),
                           preferred_element_type=f32)
    gi = lax.broadcasted_iota(jnp.int32, (TB, NLOC), 1)
    oh_lat = jnp.where(gi == loc_ref[:, 0:1], 1.0, 0.0)
    oh_lon = jnp.where(gi == loc_ref[:, 1:2], 1.0, 0.0)
    wgeo = wgeo + lax.dot_general(oh_lat, lat2_ref[:, :], (((1,), (0,)), ((), ())),
                                  preferred_element_type=f32)
    wgeo = wgeo + lax.dot_general(oh_lon, lon2_ref[:, :], (((1,), (0,)), ((), ())),
                                  preferred_element_type=f32)
    wgeo = wgeo + ball_ref[:, :]

    pred_ref[:, :] = wgeo[:, 0:1] * xx + wgeo[:, 1:2] * dist_ref[:, :]


def kernel(text_left, text_right, location_left, distance, emb_table, attn_w,
           mlp_w1, mlp_b1, mlp_w2, mlp_b2, mlp_w3, mlp_b3, out_w, out_b,
           lat_table, lon_table, lin5_w, lin5_b, lin3_w, lin3_b, lin4_w, lin4_b):
    B, LQ = text_left.shape
    LR = text_right.shape[1]
    D = emb_table.shape[1]
    NLOC = lat_table.shape[0]
    H3 = lin3_w.shape[0]
    GEOW = lin5_w.shape[0]
    f32 = jnp.float32

    tl_pad = jnp.pad(text_left, ((0, 0), (0, LQP - LQ)))

    NS = 4
    Bs = B // NS
    gathered = []
    for si in range(NS):
        idx_s = jnp.concatenate(
            [tl_pad[si * Bs:(si + 1) * Bs].reshape(-1),
             text_right[si * Bs:(si + 1) * Bs].reshape(-1)]).astype(jnp.int32)
        gathered.append(
            _gather_rows_sc(emb_table, idx_s, (Bs * LQP, Bs * LR)))

    attn_wT = attn_w.T.astype(f32)
    w1T = mlp_w1.T.astype(f32)
    b1 = mlp_b1.reshape(1, -1).astype(f32)
    w2T = mlp_w2.T.astype(f32)
    b2 = mlp_b2.reshape(1, -1).astype(f32)
    w3T = mlp_w3.T.astype(f32)
    b3 = mlp_b3.reshape(1, 1).astype(f32)
    outw = out_w.reshape(1, 1).astype(f32)
    outb = out_b.reshape(1, 1).astype(f32)

    A1 = lin4_w[:, :LQ * H3].reshape(2, LQ, H3)
    M3 = jnp.einsum('jd,clj->cld', lin3_w, A1)
    M3 = jnp.pad(M3, ((0, 0), (0, LQP - LQ), (0, 0)))
    M2 = M3.transpose(2, 0, 1).reshape(D, 2 * LQP).astype(f32)
    A2 = lin4_w[:, LQ * H3:LQ * H3 + GEOW]
    A3 = lin4_w[:, LQ * H3 + GEOW:LQ * H3 + 2 * GEOW]
    lat2 = (lat_table @ (lin5_w.T @ A2.T)).astype(f32)
    lon2 = (lon_table @ (lin5_w.T @ A3.T)).astype(f32)
    b_all = (lin4_b
             + jnp.einsum('j,clj->c', lin3_b, A1)
             + lin5_b @ A2.T + lin5_b @ A3.T).reshape(1, 2).astype(f32)

    tlf = tl_pad.astype(f32).reshape(B * LQP, 1)
    locf = location_left.astype(jnp.int32)
    dist2 = distance.astype(f32).reshape(B, 1)

    TB = min(64, Bs)
    G = Bs // TB

    body = functools.partial(_tc_body, TB=TB, LR=LR, NLOC=NLOC)

    in_specs = [
            pl.BlockSpec((TB * LQP, D), lambda g: (g, 0)),
            pl.BlockSpec((TB * LR, D), lambda g: (g, 0)),
            pl.BlockSpec((TB * LQP, 1), lambda g: (g, 0)),
            pl.BlockSpec((TB, 2), lambda g: (g, 0)),
            pl.BlockSpec((TB, 1), lambda g: (g, 0)),
            pl.BlockSpec((D, 1), lambda g: (0, 0)),
            pl.BlockSpec((K, 128), lambda g: (0, 0)),
            pl.BlockSpec((1, 128), lambda g: (0, 0)),
            pl.BlockSpec((128, 128), lambda g: (0, 0)),
            pl.BlockSpec((1, 128), lambda g: (0, 0)),
            pl.BlockSpec((128, 1), lambda g: (0, 0)),
            pl.BlockSpec((1, 1), lambda g: (0, 0)),
            pl.BlockSpec((1, 1), lambda g: (0, 0)),
            pl.BlockSpec((1, 1), lambda g: (0, 0)),
            pl.BlockSpec((D, 2 * LQP), lambda g: (0, 0)),
            pl.BlockSpec((NLOC, 2), lambda g: (0, 0)),
            pl.BlockSpec((NLOC, 2), lambda g: (0, 0)),
            pl.BlockSpec((1, 2), lambda g: (0, 0)),
    ]

    preds = []
    for si in range(NS):
        q_pad, d_rows = gathered[si]
        preds.append(pl.pallas_call(
            body,
            grid=(G,),
            in_specs=in_specs,
            out_specs=pl.BlockSpec((TB, 1), lambda g: (g, 0)),
            out_shape=jax.ShapeDtypeStruct((Bs, 1), f32),
            scratch_shapes=[
                pltpu.VMEM((TB * LQP, LR), f32),
                pltpu.VMEM((TB * LQP, K), f32),
            ],
        )(q_pad, d_rows,
          tlf[si * Bs * LQP:(si + 1) * Bs * LQP],
          locf[si * Bs:(si + 1) * Bs],
          dist2[si * Bs:(si + 1) * Bs],
          attn_wT, w1T, b1, w2T, b2,
          w3T, b3, outw, outb, M2, lat2, lon2, b_all))

    return jnp.concatenate(preds, axis=0)
